# baseline (speedup 1.0000x reference)
import jax
import jax.numpy as jnp
from jax.experimental import pallas as pl
from jax.experimental.pallas import tpu as pltpu

_H = 1000
_CC = 40


def _body(x_ref, o_ref):
    idx = x_ref[0, 0, :]
    b = idx.shape[0]
    c0 = pl.program_id(1) * _CC
    iota = c0 + jax.lax.broadcasted_iota(jnp.int32, (_CC, b), 0)
    o_ref[0] = (idx[None, :] == iota).astype(jnp.float32)


def kernel(x):
    b, s = x.shape
    xt = x.T.reshape(s, 1, b).astype(jnp.int32)
    out = pl.pallas_call(
        _body,
        grid=(s, _H // _CC),
        in_specs=[pl.BlockSpec((1, 1, b), lambda j, c: (j, 0, 0))],
        out_specs=pl.BlockSpec((1, _CC, b), lambda j, c: (j, c, 0)),
        out_shape=jax.ShapeDtypeStruct((s, _H, b), jnp.float32),
    )(xt)
    return jnp.transpose(out, (2, 0, 1))
